# baseline (speedup 1.0000x reference)
import functools
import jax
import jax.numpy as jnp
from jax import lax
from jax.experimental import pallas as pl
from jax.experimental.pallas import tpu as pltpu
from jax.experimental.pallas import tpu_sc as plsc

T, B, N = 64, 8192, 64
BSUB = 4


def _bc16(v):
    return lax.broadcast_in_dim(jnp.int32(v), (16,), ())


def kernel(x, center, scaling):
    info = plsc.get_sparse_core_info()
    nc, ns = info.num_cores, info.num_subcores
    nw = nc * ns
    bpt = B // nw
    nslab = bpt // BSUB

    @functools.partial(
        pl.kernel,
        mesh=plsc.VectorSubcoreMesh(core_axis_name="c", subcore_axis_name="s"),
        out_type=jax.ShapeDtypeStruct((T, B * N), jnp.float32),
        scratch_types=[
            pltpu.VMEM((bpt,), jnp.float32),
            pltpu.VMEM((N,), jnp.float32),
            pltpu.VMEM((N,), jnp.float32),
            pltpu.VMEM((bpt * N,), jnp.int32),
            pltpu.VMEM((2, T, BSUB * N), jnp.float32),
            pltpu.SemaphoreType.DMA,
            pltpu.SemaphoreType.DMA,
        ],
        compiler_params=pltpu.CompilerParams(needs_layout_passes=False),
    )
    def run(x_hbm, c_hbm, s_hbm, out_hbm, x_v, c_v, s_v, tsp_v, slab_v,
            sem0, sem1):
        sems = (sem0, sem1)
        wid = lax.axis_index("s") * nc + lax.axis_index("c")
        b0 = wid * bpt
        pltpu.sync_copy(x_hbm.at[pl.ds(b0, bpt)], x_v)
        pltpu.sync_copy(c_hbm, c_v)
        pltpu.sync_copy(s_hbm, s_v)

        cw = [c_v[pl.ds(nv * 16, 16)] for nv in range(4)]
        sw = [s_v[pl.ds(nv * 16, 16)] for nv in range(4)]
        iota = lax.iota(jnp.int32, 16)
        ones = jnp.ones((16,), jnp.float32)
        zeros = jnp.zeros((16,), jnp.float32)

        def tsp_body(j, carry):
            xj = plsc.load_gather(x_v, [_bc16(j)])
            for nv in range(4):
                d = sw[nv] * jnp.abs(xj - cw[nv])
                t = jnp.clip(d.astype(jnp.int32), 0, T - 1)
                tsp_v[pl.ds(j * N + nv * 16, 16)] = t
            return carry

        lax.fori_loop(0, bpt, tsp_body, 0)

        def zero_body(t, carry):
            for k in range(2):
                for j in range(BSUB * N // 16):
                    slab_v[k, t, pl.ds(j * 16, 16)] = zeros
            return carry

        lax.fori_loop(0, T, zero_body, 0)

        def scatter_slab(k, s, val):
            for bs in range(BSUB):
                for nv in range(4):
                    off = (s * BSUB + bs) * N + nv * 16
                    tvec = tsp_v[pl.ds(off, 16)]
                    plsc.store_scatter(
                        slab_v,
                        [_bc16(k), tvec, _bc16(bs * N + nv * 16) + iota],
                        val,
                    )

        def copy_op(k, s):
            return pltpu.make_async_copy(
                slab_v.at[k],
                out_hbm.at[:, pl.ds((b0 + s * BSUB) * N, BSUB * N)],
                sems[k],
            )

        for k in range(2):
            scatter_slab(k, k, ones)
            copy_op(k, k).start()

        def pair_body(i, carry):
            for k in range(2):
                s = 2 * i + k
                copy_op(k, s - 2).wait()
                scatter_slab(k, s - 2, zeros)
                scatter_slab(k, s, ones)
                copy_op(k, s).start()
            return carry

        lax.fori_loop(1, nslab // 2, pair_body, 0)

        for k in range(2):
            copy_op(k, nslab - 2 + k).wait()

    return run(x, center, scaling).reshape(T, B, N)


# speedup vs baseline: 1.5317x; 1.5317x over previous
import jax
import jax.numpy as jnp
from jax import lax
from jax.experimental import pallas as pl

TIME_STEPS = 64
T_BLK = 1


def _onehot_kernel(x_ref, c_ref, s_ref, out_ref):
    i = pl.program_id(0)
    t_base = i * T_BLK
    xv = x_ref[:][:, None]
    cv = c_ref[:][None, :]
    sv = s_ref[:][None, :]
    dist = sv * jnp.abs(xv - cv)
    tsp = jnp.clip(dist.astype(jnp.int32), 0, TIME_STEPS - 1)
    shape = out_ref.shape
    t_ids = lax.broadcasted_iota(jnp.int32, shape, 0) + t_base
    out_ref[:] = (t_ids == tsp[None, :, :]).astype(jnp.float32)


def kernel(x, center, scaling):
    b = x.shape[0]
    n = center.shape[0]
    grid = (TIME_STEPS // T_BLK,)
    return pl.pallas_call(
        _onehot_kernel,
        grid=grid,
        in_specs=[
            pl.BlockSpec((b,), lambda i: (0,)),
            pl.BlockSpec((n,), lambda i: (0,)),
            pl.BlockSpec((n,), lambda i: (0,)),
        ],
        out_specs=pl.BlockSpec((T_BLK, b, n), lambda i: (i, 0, 0)),
        out_shape=jax.ShapeDtypeStruct((TIME_STEPS, b, n), jnp.float32),
    )(x, center, scaling)


# speedup vs baseline: 1.7139x; 1.1189x over previous
import jax
import jax.numpy as jnp
from jax import lax
from jax.experimental import pallas as pl
from jax.experimental.pallas import tpu as pltpu

TIME_STEPS = 64
T_BLK = 2


def _onehot_kernel(x_ref, c_ref, s_ref, out_ref, tsp_ref):
    i = pl.program_id(0)

    @pl.when(i == 0)
    def _compute_tsp():
        xv = x_ref[:][:, None]
        cv = c_ref[:][None, :]
        sv = s_ref[:][None, :]
        dist = sv * jnp.abs(xv - cv)
        tsp_ref[:] = jnp.clip(dist.astype(jnp.int32), 0, TIME_STEPS - 1)

    t_base = i * T_BLK
    tsp = tsp_ref[:]
    shape = out_ref.shape
    t_ids = lax.broadcasted_iota(jnp.int32, shape, 0) + t_base
    out_ref[:] = (t_ids == tsp[None, :, :]).astype(jnp.float32)


def kernel(x, center, scaling):
    b = x.shape[0]
    n = center.shape[0]
    grid = (TIME_STEPS // T_BLK,)
    return pl.pallas_call(
        _onehot_kernel,
        grid=grid,
        in_specs=[
            pl.BlockSpec((b,), lambda i: (0,)),
            pl.BlockSpec((n,), lambda i: (0,)),
            pl.BlockSpec((n,), lambda i: (0,)),
        ],
        out_specs=pl.BlockSpec((T_BLK, b, n), lambda i: (i, 0, 0)),
        out_shape=jax.ShapeDtypeStruct((TIME_STEPS, b, n), jnp.float32),
        scratch_shapes=[pltpu.VMEM((b, n), jnp.int32)],
    )(x, center, scaling)
